# Initial kernel scaffold; baseline (speedup 1.0000x reference)
import jax
import jax.numpy as jnp
from jax import lax
from jax.experimental import pallas as pl
from jax.experimental.pallas import tpu as pltpu

D = 1024
H = 16
HD = D // H
B, S = 2, 1024
N = B * S
BQ = 256
SCALE = 1.0 / 32.0

_CONTRACT_LAST = (((1,), (1,)), ((), ()))


def _mha_kernel(x_ref, w_ref, b_ref, wq_ref, wk_ref, o_ref):
    x = x_ref[...]
    w = w_ref[...]
    flat = lax.dot_general(x, w, _CONTRACT_LAST,
                           preferred_element_type=jnp.float32) + b_ref[0]
    q = jnp.dot(flat, wq_ref[0], preferred_element_type=jnp.float32)
    k = jnp.dot(flat, wk_ref[0], preferred_element_type=jnp.float32)
    for i in range(N // BQ):
        qi = q[i * BQ:(i + 1) * BQ, :]
        s = lax.dot_general(qi, k, _CONTRACT_LAST,
                            preferred_element_type=jnp.float32) * SCALE
        m = jnp.max(s, axis=-1, keepdims=True)
        p = jnp.exp(s - m)
        p = p / jnp.sum(p, axis=-1, keepdims=True)
        o_ref[i * BQ:(i + 1) * BQ, :] = jnp.dot(
            p, flat, preferred_element_type=jnp.float32)


def kernel(x, proj_w, proj_b, wq, wk):
    x2d = x.reshape(N, D)
    b3d = proj_b.reshape(H, 1, HD)
    out = pl.pallas_call(
        _mha_kernel,
        grid=(H,),
        in_specs=[
            pl.BlockSpec((N, D), lambda h: (0, 0)),
            pl.BlockSpec((HD, D), lambda h: (h, 0)),
            pl.BlockSpec((1, 1, HD), lambda h: (h, 0, 0)),
            pl.BlockSpec((1, HD, HD), lambda h: (h, 0, 0)),
            pl.BlockSpec((1, HD, HD), lambda h: (h, 0, 0)),
        ],
        out_specs=pl.BlockSpec((N, HD), lambda h: (0, h)),
        out_shape=jax.ShapeDtypeStruct((N, D), jnp.float32),
        compiler_params=pltpu.CompilerParams(
            dimension_semantics=("parallel",),
            vmem_limit_bytes=100 * 1024 * 1024,
        ),
    )(x2d, proj_w, b3d, wq, wk)
    return out.reshape(B, S, D)


# speedup vs baseline: 1.8839x; 1.8839x over previous
import jax
import jax.numpy as jnp
from jax import lax
from jax.experimental import pallas as pl
from jax.experimental.pallas import tpu as pltpu

D = 1024
H = 16
HD = D // H
B, S = 2, 1024
N = B * S
PAIR = 2
GH = H // PAIR
BQ = 256
SCALE = 1.0 / 32.0

_CONTRACT_LAST = (((1,), (1,)), ((), ()))


def _mha_kernel(x_ref, w_ref, b_ref, wq_ref, wk_ref, o_ref):
    x = x_ref[...]
    w = w_ref[...]
    flat2 = lax.dot_general(x, w, _CONTRACT_LAST,
                            preferred_element_type=jnp.float32) + b_ref[0]
    for p in range(PAIR):
        flat = flat2[:, p * HD:(p + 1) * HD]
        q = jnp.dot(flat, wq_ref[p], preferred_element_type=jnp.float32)
        k = jnp.dot(flat, wk_ref[p], preferred_element_type=jnp.float32)
        for i in range(N // BQ):
            qi = q[i * BQ:(i + 1) * BQ, :]
            s = lax.dot_general(qi, k, _CONTRACT_LAST,
                                preferred_element_type=jnp.float32) * SCALE
            m = jnp.max(s, axis=-1, keepdims=True)
            e = jnp.exp(s - m)
            pr = e / jnp.sum(e, axis=-1, keepdims=True)
            o_ref[i * BQ:(i + 1) * BQ, p * HD:(p + 1) * HD] = jnp.dot(
                pr, flat, preferred_element_type=jnp.float32)


def kernel(x, proj_w, proj_b, wq, wk):
    x2d = x.reshape(N, D)
    b3d = proj_b.reshape(GH, 1, PAIR * HD)
    out = pl.pallas_call(
        _mha_kernel,
        grid=(GH,),
        in_specs=[
            pl.BlockSpec((N, D), lambda g: (0, 0)),
            pl.BlockSpec((PAIR * HD, D), lambda g: (g, 0)),
            pl.BlockSpec((1, 1, PAIR * HD), lambda g: (g, 0, 0)),
            pl.BlockSpec((PAIR, HD, HD), lambda g: (g, 0, 0)),
            pl.BlockSpec((PAIR, HD, HD), lambda g: (g, 0, 0)),
        ],
        out_specs=pl.BlockSpec((N, PAIR * HD), lambda g: (0, g)),
        out_shape=jax.ShapeDtypeStruct((N, D), jnp.float32),
        compiler_params=pltpu.CompilerParams(
            dimension_semantics=("parallel",),
            vmem_limit_bytes=56 * 1024 * 1024,
        ),
    )(x2d, proj_w, b3d, wq, wk)
    return out.reshape(B, S, D)
